# baseline (speedup 1.0000x reference)
import jax
import jax.numpy as jnp
from jax.experimental import pallas as pl


_BL = 256


def _add_kernel(x_ref, pe_ref, o_ref):
    o_ref[...] = x_ref[...] + pe_ref[...]


def kernel(x, pos_embedding):
    if x.ndim != 3:
        raise ValueError(
            f'Expected input to have 3 dimensions, but got {x.ndim} dimensions')
    B, L, D = x.shape
    pe = pos_embedding[:L]
    grid = (L // _BL, B)
    return pl.pallas_call(
        _add_kernel,
        grid=grid,
        in_specs=[
            pl.BlockSpec((1, _BL, D), lambda l, b: (b, l, 0)),
            pl.BlockSpec((_BL, D), lambda l, b: (l, 0)),
        ],
        out_specs=pl.BlockSpec((1, _BL, D), lambda l, b: (b, l, 0)),
        out_shape=jax.ShapeDtypeStruct((B, L, D), x.dtype),
    )(x, pe)


# speedup vs baseline: 1.4285x; 1.4285x over previous
import jax
import jax.numpy as jnp
from jax.experimental import pallas as pl


_BL = 1024


def _add_kernel(x_ref, pe_ref, o_ref):
    o_ref[...] = x_ref[...] + pe_ref[...]


def kernel(x, pos_embedding):
    if x.ndim != 3:
        raise ValueError(
            f'Expected input to have 3 dimensions, but got {x.ndim} dimensions')
    B, L, D = x.shape
    pe = pos_embedding[:L]
    grid = (L // _BL, B)
    return pl.pallas_call(
        _add_kernel,
        grid=grid,
        in_specs=[
            pl.BlockSpec((1, _BL, D), lambda l, b: (b, l, 0)),
            pl.BlockSpec((_BL, D), lambda l, b: (l, 0)),
        ],
        out_specs=pl.BlockSpec((1, _BL, D), lambda l, b: (b, l, 0)),
        out_shape=jax.ShapeDtypeStruct((B, L, D), x.dtype),
    )(x, pe)


# speedup vs baseline: 1.5492x; 1.0845x over previous
import jax
import jax.numpy as jnp
from jax.experimental import pallas as pl


_BL = 2048


def _add_kernel(x_ref, pe_ref, o_ref):
    o_ref[...] = x_ref[...] + pe_ref[...]


def kernel(x, pos_embedding):
    if x.ndim != 3:
        raise ValueError(
            f'Expected input to have 3 dimensions, but got {x.ndim} dimensions')
    B, L, D = x.shape
    pe = pos_embedding[:L]
    grid = (L // _BL, B)
    return pl.pallas_call(
        _add_kernel,
        grid=grid,
        in_specs=[
            pl.BlockSpec((1, _BL, D), lambda l, b: (b, l, 0)),
            pl.BlockSpec((_BL, D), lambda l, b: (l, 0)),
        ],
        out_specs=pl.BlockSpec((1, _BL, D), lambda l, b: (b, l, 0)),
        out_shape=jax.ShapeDtypeStruct((B, L, D), x.dtype),
    )(x, pe)
